# baseline (speedup 1.0000x reference)
import functools

import jax
import jax.numpy as jnp
from jax.experimental import pallas as pl
from jax.experimental.pallas import tpu as pltpu

_K = 10
_QB = 512
_KB = 16384
_LANES = 128
_BIG = 2**30


def _knn_block_kernel(xt_ref, xtr_ref, vals_ref, idx_ref, sref, cval, cidx,
                      *, n_total, kb, nsteps, k):
    ni = pl.program_id(1)

    @pl.when(ni == 0)
    def _init():
        cval[...] = jnp.full(cval.shape, -jnp.inf, dtype=cval.dtype)
        cidx[...] = jnp.zeros(cidx.shape, dtype=cidx.dtype)

    a = xt_ref[...]
    bt = xtr_ref[...]
    b2 = jnp.sum(bt * bt, axis=0)
    ab = jax.lax.dot_general(a, bt, (((1,), (0,)), ((), ())),
                             preferred_element_type=jnp.float32)
    base = ni * kb
    col0 = jax.lax.broadcasted_iota(jnp.int32, ab.shape, 1)
    s_new = jnp.where(col0 + base < n_total,
                      b2[None, :] - 2.0 * ab, -jnp.inf)
    sref[...] = s_new

    lane0 = jax.lax.broadcasted_iota(jnp.int32, cval.shape, 1)
    tau = jnp.max(jnp.where(lane0 == k - 1, cval[...], -jnp.inf),
                  axis=1, keepdims=True)
    cnt = jnp.sum((s_new > tau).astype(jnp.int32), axis=1)
    mx = jnp.max(cnt)
    n_iter = jnp.minimum(mx, k)

    def body1(t, carry):
        s = sref[...]
        col = jax.lax.broadcasted_iota(jnp.int32, s.shape, 1)
        lane = jax.lax.broadcasted_iota(jnp.int32, cval.shape, 1)
        m = jnp.max(s, axis=1, keepdims=True)
        eq = s == m
        am = jnp.min(jnp.where(eq, col, _BIG), axis=1, keepdims=True)
        cval[...] = jnp.where(lane == 31 - t, m, cval[...])
        cidx[...] = jnp.where(lane == 31 - t, am + base, cidx[...])
        sref[...] = jnp.where(col == am, -jnp.inf, s)
        return carry

    @pl.when(mx > 0)
    def _select():
        lane = jax.lax.broadcasted_iota(jnp.int32, cval.shape, 1)
        cval[...] = jnp.where(lane < k, cval[...], -jnp.inf)
        jax.lax.fori_loop(0, n_iter, body1, 0, unroll=1)
        v = cval[...]
        i = cidx[...]
        for d in (16, 8, 4, 2, 1):
            low = (lane & d) == 0
            pv = jnp.where(low, jnp.roll(v, -d, axis=1), jnp.roll(v, d, axis=1))
            pi = jnp.where(low, jnp.roll(i, -d, axis=1), jnp.roll(i, d, axis=1))
            lex_gt = (v > pv) | ((v == pv) & (i < pi))
            hi_v = jnp.where(lex_gt, v, pv)
            hi_i = jnp.where(lex_gt, i, pi)
            lo_v = jnp.where(lex_gt, pv, v)
            lo_i = jnp.where(lex_gt, pi, i)
            v = jnp.where(low, hi_v, lo_v)
            i = jnp.where(low, hi_i, lo_i)
        cval[...] = v
        cidx[...] = i

    @pl.when(ni == nsteps - 1)
    def _finish():
        a2 = jnp.sum(a * a, axis=1, keepdims=True)
        d2 = jnp.maximum(a2 + cval[...], 0.0)
        vals_ref[...] = jnp.sqrt(d2)
        idx_ref[...] = cidx[...]


@functools.partial(jax.jit, static_argnames=("k", "qb", "kb"))
def _knn_topk(x_test, x_train, k=_K, qb=_QB, kb=_KB):
    q, d = x_test.shape
    n = x_train.shape[0]
    nsteps = -(-n // kb)
    n_pad = nsteps * kb
    xtr_t = x_train.T
    if n_pad != n:
        xtr_t = jnp.pad(xtr_t, ((0, 0), (0, n_pad - n)))
    grid = (q // qb, nsteps)
    kern = functools.partial(_knn_block_kernel, n_total=n, kb=kb,
                             nsteps=nsteps, k=k)
    vals, idx = pl.pallas_call(
        kern,
        grid=grid,
        in_specs=[
            pl.BlockSpec((qb, d), lambda qi, ni: (qi, 0)),
            pl.BlockSpec((d, kb), lambda qi, ni: (0, ni)),
        ],
        out_specs=[
            pl.BlockSpec((qb, _LANES), lambda qi, ni: (qi, 0)),
            pl.BlockSpec((qb, _LANES), lambda qi, ni: (qi, 0)),
        ],
        out_shape=[
            jax.ShapeDtypeStruct((q, _LANES), jnp.float32),
            jax.ShapeDtypeStruct((q, _LANES), jnp.int32),
        ],
        scratch_shapes=[
            pltpu.VMEM((qb, kb), jnp.float32),
            pltpu.VMEM((qb, _LANES), jnp.float32),
            pltpu.VMEM((qb, _LANES), jnp.int32),
        ],
        compiler_params=pltpu.CompilerParams(
            dimension_semantics=("parallel", "arbitrary"),
        ),
    )(x_test, xtr_t)
    return vals[:, :k], idx[:, :k]


def kernel(x_test, x_train, y_train):
    return _knn_topk(x_test, x_train)


# speedup vs baseline: 1.5532x; 1.5532x over previous
import functools

import jax
import jax.numpy as jnp
from jax.experimental import pallas as pl
from jax.experimental.pallas import tpu as pltpu

_K = 10
_QB = 512
_KB = 8192
_LANES = 128
_BIG = 2**30


def _knn_block_kernel(xt_ref, xtr_ref, vals_ref, idx_ref, sref, cval, cidx,
                      *, n_total, kb, nsteps, k):
    ni = pl.program_id(1)

    @pl.when(ni == 0)
    def _init():
        cval[...] = jnp.full(cval.shape, -jnp.inf, dtype=cval.dtype)
        cidx[...] = jnp.zeros(cidx.shape, dtype=cidx.dtype)

    a = xt_ref[...]
    bt = xtr_ref[...]
    b2 = jnp.sum(bt * bt, axis=0)
    ab = jax.lax.dot_general(a, bt, (((1,), (0,)), ((), ())),
                             preferred_element_type=jnp.float32)
    base = ni * kb
    col0 = jax.lax.broadcasted_iota(jnp.int32, ab.shape, 1)
    s_new = jnp.where(col0 + base < n_total,
                      b2[None, :] - 2.0 * ab, -jnp.inf)
    sref[...] = s_new

    sub_w = min(kb, 2048)

    def _process_sub(lo):
        lane0 = jax.lax.broadcasted_iota(jnp.int32, cval.shape, 1)
        tau = jnp.max(jnp.where(lane0 == k - 1, cval[...], -jnp.inf),
                      axis=1, keepdims=True)
        s_sub = sref[:, lo:lo + sub_w]
        cnt = jnp.sum((s_sub > tau).astype(jnp.int32), axis=1)
        mx = jnp.max(cnt)
        n_iter = jnp.minimum(mx, k)

        def body1(t, carry):
            s = sref[:, lo:lo + sub_w]
            col = jax.lax.broadcasted_iota(jnp.int32, s.shape, 1)
            lane = jax.lax.broadcasted_iota(jnp.int32, cval.shape, 1)
            m = jnp.max(s, axis=1, keepdims=True)
            eq = s == m
            am = jnp.min(jnp.where(eq, col, _BIG), axis=1, keepdims=True)
            cval[...] = jnp.where(lane == 31 - t, m, cval[...])
            cidx[...] = jnp.where(lane == 31 - t, am + (base + lo), cidx[...])
            sref[:, lo:lo + sub_w] = jnp.where(col == am, -jnp.inf, s)
            return carry

        @pl.when(mx > 0)
        def _select():
            lane = jax.lax.broadcasted_iota(jnp.int32, cval.shape, 1)
            cval[...] = jnp.where(lane < k, cval[...], -jnp.inf)
            jax.lax.fori_loop(0, n_iter, body1, 0, unroll=1)
            v = cval[...]
            i = cidx[...]
            for d in (16, 8, 4, 2, 1):
                low = (lane & d) == 0
                pv = jnp.where(low, jnp.roll(v, -d, axis=1),
                               jnp.roll(v, d, axis=1))
                pi = jnp.where(low, jnp.roll(i, -d, axis=1),
                               jnp.roll(i, d, axis=1))
                lex_gt = (v > pv) | ((v == pv) & (i < pi))
                hi_v = jnp.where(lex_gt, v, pv)
                hi_i = jnp.where(lex_gt, i, pi)
                lo_v = jnp.where(lex_gt, pv, v)
                lo_i = jnp.where(lex_gt, pi, i)
                v = jnp.where(low, hi_v, lo_v)
                i = jnp.where(low, hi_i, lo_i)
            cval[...] = v
            cidx[...] = i

    for _lo in range(0, kb, sub_w):
        _process_sub(_lo)

    @pl.when(ni == nsteps - 1)
    def _finish():
        a2 = jnp.sum(a * a, axis=1, keepdims=True)
        d2 = jnp.maximum(a2 + cval[...], 0.0)
        vals_ref[...] = jnp.sqrt(d2)
        idx_ref[...] = cidx[...]


@functools.partial(jax.jit, static_argnames=("k", "qb", "kb"))
def _knn_topk(x_test, x_train, k=_K, qb=_QB, kb=_KB):
    q, d = x_test.shape
    n = x_train.shape[0]
    nsteps = -(-n // kb)
    n_pad = nsteps * kb
    xtr_t = x_train.T
    if n_pad != n:
        xtr_t = jnp.pad(xtr_t, ((0, 0), (0, n_pad - n)))
    grid = (q // qb, nsteps)
    kern = functools.partial(_knn_block_kernel, n_total=n, kb=kb,
                             nsteps=nsteps, k=k)
    vals, idx = pl.pallas_call(
        kern,
        grid=grid,
        in_specs=[
            pl.BlockSpec((qb, d), lambda qi, ni: (qi, 0)),
            pl.BlockSpec((d, kb), lambda qi, ni: (0, ni)),
        ],
        out_specs=[
            pl.BlockSpec((qb, _LANES), lambda qi, ni: (qi, 0)),
            pl.BlockSpec((qb, _LANES), lambda qi, ni: (qi, 0)),
        ],
        out_shape=[
            jax.ShapeDtypeStruct((q, _LANES), jnp.float32),
            jax.ShapeDtypeStruct((q, _LANES), jnp.int32),
        ],
        scratch_shapes=[
            pltpu.VMEM((qb, kb), jnp.float32),
            pltpu.VMEM((qb, _LANES), jnp.float32),
            pltpu.VMEM((qb, _LANES), jnp.int32),
        ],
        compiler_params=pltpu.CompilerParams(
            dimension_semantics=("parallel", "arbitrary"),
        ),
    )(x_test, xtr_t)
    return vals[:, :k], idx[:, :k]


def kernel(x_test, x_train, y_train):
    return _knn_topk(x_test, x_train)
